# baseline (speedup 1.0000x reference)
import functools

import jax
import jax.numpy as jnp
from jax import lax
from jax.experimental import pallas as pl
from jax.experimental.pallas import tpu as pltpu, tpu_sc as plsc

N_EMB = 1024
D = 64
B = 32
L = 576
_LOSS_SCALE = 1.25 / (B * L * D)

_NL = 16
_CH = 96
_CW = 128


def _vq_tc_body(z_ref, emb_ref, loss_ref, codes_ref):
    b = pl.program_id(0)
    z = z_ref[0]
    emb = emb_ref[...]

    z_sq = jnp.sum(z * z, axis=0, keepdims=True)
    e_sq = jnp.sum(emb * emb, axis=1, keepdims=True)
    m2 = jax.lax.dot_general(
        emb, z + z, (((1,), (0,)), ((), ())),
        preferred_element_type=jnp.float32)
    dist = (z_sq - m2) + e_sq

    min_d = jnp.min(dist, axis=0, keepdims=True)
    k_iota = jax.lax.broadcasted_iota(jnp.int32, dist.shape, 0)
    codes_ref[0] = jnp.min(jnp.where(dist == min_d, k_iota, N_EMB),
                           axis=0, keepdims=True)

    part = jnp.sum(min_d, axis=1, keepdims=True)

    @pl.when(b == 0)
    def _():
        loss_ref[...] = jnp.zeros((1, 1), jnp.float32)

    total = loss_ref[...] + part
    loss_ref[...] = jnp.where(b == B - 1, total * _LOSS_SCALE, total)


def _codes_and_loss(z, embedding):
    loss, codes3 = pl.pallas_call(
        _vq_tc_body,
        grid=(B,),
        in_specs=[
            pl.BlockSpec((1, D, L), lambda b: (b, 0, 0)),
            pl.BlockSpec((N_EMB, D), lambda b: (0, 0)),
        ],
        out_specs=[
            pl.BlockSpec((1, 1), lambda b: (0, 0)),
            pl.BlockSpec((1, 1, L), lambda b: (b, 0, 0)),
        ],
        out_shape=[
            jax.ShapeDtypeStruct((1, 1), jnp.float32),
            jax.ShapeDtypeStruct((B, 1, L), jnp.int32),
        ],
    )(z, embedding)
    return loss[0, 0], codes3.reshape(B, L)


def _sc_gather_body(emb_hbm, codes_hbm, zq_hbm,
                    codes_v, gidx_v, off_v, rows_v, out_v, sem):
    info = plsc.get_sparse_core_info()
    wid = lax.axis_index("s") * info.num_cores + lax.axis_index("c")

    pltpu.sync_copy(codes_hbm.at[wid], codes_v)
    for i in range(L // _NL):
        c16 = codes_v[pl.ds(i * _NL, _NL)]
        gidx_v[pl.ds(i * _NL, _NL)] = c16 >> 1
        off_v[pl.ds(i * _NL, _NL)] = (c16 & 1) << 6

    copies = [
        pltpu.async_copy(
            emb_hbm.at[gidx_v.at[pl.ds(j * _CH, _CH)]],
            rows_v.at[pl.ds(j * _CH, _CH), :], sem.at[j])
        for j in range(L // _CH)
    ]

    lane = lax.iota(jnp.int32, _NL)
    per_chunk = _CH // _NL

    def make_tr(base_lc):
        def tr(i, carry):
            lc = base_lc + i
            row16 = lc * _NL + lane
            col_base = off_v[pl.ds(lc * _NL, _NL)]
            for d in range(D):
                vals = plsc.load_gather(rows_v, [row16, col_base + d])
                out_v[d, pl.ds(lc * _NL, _NL)] = vals
            return carry
        return tr

    for j in range(L // _CH):
        copies[j].wait()
        lax.fori_loop(0, per_chunk, make_tr(j * per_chunk), 0)

    pltpu.sync_copy(out_v, zq_hbm.at[wid])


def _sc_gather(embedding, codes):
    mesh = plsc.VectorSubcoreMesh(core_axis_name="c", subcore_axis_name="s")
    return pl.kernel(
        _sc_gather_body,
        out_type=jax.ShapeDtypeStruct((B, D, L), jnp.float32),
        mesh=mesh,
        compiler_params=pltpu.CompilerParams(needs_layout_passes=False),
        scratch_types=[
            pltpu.VMEM((L,), jnp.int32),
            pltpu.VMEM((L,), jnp.int32),
            pltpu.VMEM((L,), jnp.int32),
            pltpu.VMEM((L, _CW), jnp.float32),
            pltpu.VMEM((D, L), jnp.float32),
            pltpu.SemaphoreType.DMA((L // _CH,)),
        ],
    )(embedding.reshape(N_EMB // 2, _CW), codes)


@jax.jit
def kernel(z, embedding):
    loss, codes = _codes_and_loss(z, embedding)
    zq = _sc_gather(embedding, codes)
    return zq, loss, codes


# speedup vs baseline: 1.1669x; 1.1669x over previous
import functools

import jax
import jax.numpy as jnp
from jax import lax
from jax.experimental import pallas as pl
from jax.experimental.pallas import tpu as pltpu, tpu_sc as plsc

N_EMB = 1024
D = 64
B = 32
L = 576
_LOSS_SCALE = 1.25 / (B * L * D)

_NL = 16
_CH = 96
_CW = 128


def _vq_tc_body(z_ref, emb_ref, loss_ref, codes_ref):
    b = pl.program_id(0)
    z = z_ref[0]
    emb = emb_ref[...]

    z_sq = jnp.sum(z * z, axis=0, keepdims=True)
    e_sq = jnp.sum(emb * emb, axis=1, keepdims=True)
    m2 = jax.lax.dot_general(
        emb, z + z, (((1,), (0,)), ((), ())),
        preferred_element_type=jnp.float32)
    dist = (z_sq - m2) + e_sq

    min_d = jnp.min(dist, axis=0, keepdims=True)
    k_iota = jax.lax.broadcasted_iota(jnp.int32, dist.shape, 0)
    codes_ref[0] = jnp.min(jnp.where(dist == min_d, k_iota, N_EMB),
                           axis=0, keepdims=True)

    part = jnp.sum(min_d, axis=1, keepdims=True)

    @pl.when(b == 0)
    def _():
        loss_ref[...] = jnp.zeros((1, 1), jnp.float32)

    total = loss_ref[...] + part
    loss_ref[...] = jnp.where(b == B - 1, total * _LOSS_SCALE, total)


def _codes_and_loss(z, embedding):
    loss, codes3 = pl.pallas_call(
        _vq_tc_body,
        grid=(B,),
        in_specs=[
            pl.BlockSpec((1, D, L), lambda b: (b, 0, 0)),
            pl.BlockSpec((N_EMB, D), lambda b: (0, 0)),
        ],
        out_specs=[
            pl.BlockSpec((1, 1), lambda b: (0, 0)),
            pl.BlockSpec((1, 1, L), lambda b: (b, 0, 0)),
        ],
        out_shape=[
            jax.ShapeDtypeStruct((1, 1), jnp.float32),
            jax.ShapeDtypeStruct((B, 1, L), jnp.int32),
        ],
    )(z, embedding)
    return loss[0, 0], codes3.reshape(B, L)


def _sc_gather_body(emb_hbm, codes_hbm, zq_hbm,
                    codes_v, gidx_v, off_v, rows_v, out_v, sem):
    info = plsc.get_sparse_core_info()
    wid = lax.axis_index("s") * info.num_cores + lax.axis_index("c")

    pltpu.sync_copy(codes_hbm.at[wid], codes_v)
    for i in range(L // _NL):
        c16 = codes_v[pl.ds(i * _NL, _NL)]
        gidx_v[pl.ds(i * _NL, _NL)] = c16 >> 1
        off_v[pl.ds(i * _NL, _NL)] = (c16 & 1) << 6

    copies = [
        pltpu.async_copy(
            emb_hbm.at[gidx_v.at[pl.ds(j * _CH, _CH)]],
            rows_v.at[pl.ds(j * _CH, _CH), :], sem.at[j])
        for j in range(L // _CH)
    ]

    lane = lax.iota(jnp.int32, _NL)
    per_chunk = _CH // _NL

    for j in range(L // _CH):
        copies[j].wait()

        @plsc.parallel_loop(0, per_chunk * D, unroll=8)
        def _(i, _j=j):
            lc = _j * per_chunk + (i >> 6)
            d = i & (D - 1)
            col = off_v[pl.ds(lc * _NL, _NL)] + d
            vals = plsc.load_gather(rows_v, [lc * _NL + lane, col])
            out_v[d, pl.ds(lc * _NL, _NL)] = vals

    pltpu.sync_copy(out_v, zq_hbm.at[wid])


def _sc_gather(embedding, codes):
    mesh = plsc.VectorSubcoreMesh(core_axis_name="c", subcore_axis_name="s")
    return pl.kernel(
        _sc_gather_body,
        out_type=jax.ShapeDtypeStruct((B, D, L), jnp.float32),
        mesh=mesh,
        compiler_params=pltpu.CompilerParams(needs_layout_passes=False),
        scratch_types=[
            pltpu.VMEM((L,), jnp.int32),
            pltpu.VMEM((L,), jnp.int32),
            pltpu.VMEM((L,), jnp.int32),
            pltpu.VMEM((L, _CW), jnp.float32),
            pltpu.VMEM((D, L), jnp.float32),
            pltpu.SemaphoreType.DMA((L // _CH,)),
        ],
    )(embedding.reshape(N_EMB // 2, _CW), codes)


@jax.jit
def kernel(z, embedding):
    loss, codes = _codes_and_loss(z, embedding)
    zq = _sc_gather(embedding, codes)
    return zq, loss, codes


# speedup vs baseline: 1.1748x; 1.0068x over previous
import functools

import jax
import jax.numpy as jnp
from jax import lax
from jax.experimental import pallas as pl
from jax.experimental.pallas import tpu as pltpu, tpu_sc as plsc

N_EMB = 1024
D = 64
B = 32
L = 576
_LOSS_SCALE = 1.25 / (B * L * D)

_NL = 16
_CH = 96
_CW = 128


def _vq_tc_body(z_ref, emb_ref, loss_ref, codes_ref):
    b = pl.program_id(0)
    z = z_ref[0]
    emb = emb_ref[...]

    z_sq = jnp.sum(z * z, axis=0, keepdims=True)
    e_sq = jnp.sum(emb * emb, axis=1, keepdims=True)
    m2 = jax.lax.dot_general(
        emb, z + z, (((1,), (0,)), ((), ())),
        preferred_element_type=jnp.float32)
    dist = (z_sq - m2) + e_sq

    min_d = jnp.min(dist, axis=0, keepdims=True)
    k_iota = jax.lax.broadcasted_iota(jnp.int32, dist.shape, 0)
    codes_ref[0] = jnp.min(jnp.where(dist == min_d, k_iota, N_EMB),
                           axis=0, keepdims=True)

    part = jnp.sum(min_d, axis=1, keepdims=True)

    @pl.when(b == 0)
    def _():
        loss_ref[...] = jnp.zeros((1, 1), jnp.float32)

    total = loss_ref[...] + part
    loss_ref[...] = jnp.where(b == B - 1, total * _LOSS_SCALE, total)


def _codes_and_loss(z, embedding):
    loss, codes3 = pl.pallas_call(
        _vq_tc_body,
        grid=(B,),
        in_specs=[
            pl.BlockSpec((1, D, L), lambda b: (b, 0, 0)),
            pl.BlockSpec((N_EMB, D), lambda b: (0, 0)),
        ],
        out_specs=[
            pl.BlockSpec((1, 1), lambda b: (0, 0)),
            pl.BlockSpec((1, 1, L), lambda b: (b, 0, 0)),
        ],
        out_shape=[
            jax.ShapeDtypeStruct((1, 1), jnp.float32),
            jax.ShapeDtypeStruct((B, 1, L), jnp.int32),
        ],
    )(z, embedding)
    return loss[0, 0], codes3.reshape(B, L)


def _sc_gather_body(emb_hbm, codes_hbm, zq_hbm,
                    codes_v, gidx_v, off_v, rows_v, out_v, sem):
    info = plsc.get_sparse_core_info()
    wid = lax.axis_index("s") * info.num_cores + lax.axis_index("c")

    pltpu.sync_copy(codes_hbm.at[wid], codes_v)
    for i in range(L // _NL):
        c16 = codes_v[pl.ds(i * _NL, _NL)]
        gidx_v[pl.ds(i * _NL, _NL)] = c16 >> 1
        off_v[pl.ds(i * _NL, _NL)] = (c16 & 1) << 6

    copies = [
        pltpu.async_copy(
            emb_hbm.at[gidx_v.at[pl.ds(j * _CH, _CH)]],
            rows_v.at[pl.ds(j * _CH, _CH), :], sem.at[j])
        for j in range(L // _CH)
    ]

    lane = lax.iota(jnp.int32, _NL)
    per_chunk = _CH // _NL

    def lc_body(lc, carry):
        row16 = lc * _NL + lane
        cbase = off_v[pl.ds(lc * _NL, _NL)]

        @plsc.parallel_loop(0, D, unroll=16)
        def _(d):
            vals = plsc.load_gather(rows_v, [row16, cbase + d])
            out_v[d, pl.ds(lc * _NL, _NL)] = vals

        return carry

    for j in range(L // _CH):
        copies[j].wait()
        lax.fori_loop(j * per_chunk, (j + 1) * per_chunk, lc_body, 0)

    pltpu.sync_copy(out_v, zq_hbm.at[wid])


def _sc_gather(embedding, codes):
    mesh = plsc.VectorSubcoreMesh(core_axis_name="c", subcore_axis_name="s")
    return pl.kernel(
        _sc_gather_body,
        out_type=jax.ShapeDtypeStruct((B, D, L), jnp.float32),
        mesh=mesh,
        compiler_params=pltpu.CompilerParams(needs_layout_passes=False),
        scratch_types=[
            pltpu.VMEM((L,), jnp.int32),
            pltpu.VMEM((L,), jnp.int32),
            pltpu.VMEM((L,), jnp.int32),
            pltpu.VMEM((L, _CW), jnp.float32),
            pltpu.VMEM((D, L), jnp.float32),
            pltpu.SemaphoreType.DMA((L // _CH,)),
        ],
    )(embedding.reshape(N_EMB // 2, _CW), codes)


@jax.jit
def kernel(z, embedding):
    loss, codes = _codes_and_loss(z, embedding)
    zq = _sc_gather(embedding, codes)
    return zq, loss, codes


# speedup vs baseline: 1.2673x; 1.0787x over previous
import jax
import jax.numpy as jnp
from jax import lax
from jax.experimental import pallas as pl
from jax.experimental.pallas import tpu as pltpu, tpu_sc as plsc

N_EMB = 1024
D = 64
B = 32
L = 576
_LOSS_SCALE = 1.25 / (B * L * D)

_NL = 16
_CH = 96
_CW = 128
_SL = 8


def _vq_tc_body(z_ref, emb_ref, loss_ref, codes_ref):
    b = pl.program_id(0)
    z = z_ref[0]
    emb = emb_ref[...]

    z_sq = jnp.sum(z * z, axis=0, keepdims=True)
    e_sq = jnp.sum(emb * emb, axis=1, keepdims=True)
    m2 = jax.lax.dot_general(
        emb, z + z, (((1,), (0,)), ((), ())),
        preferred_element_type=jnp.float32)

    kio = jax.lax.broadcasted_iota(jnp.int32, (_SL, L), 0)
    minv = (z_sq - m2[0:_SL]) + e_sq[0:_SL]
    mini = kio
    for g in range(1, N_EMB // _SL):
        dch = (z_sq - m2[g * _SL:(g + 1) * _SL]) + e_sq[g * _SL:(g + 1) * _SL]
        upd = dch < minv
        minv = jnp.where(upd, dch, minv)
        mini = jnp.where(upd, kio + g * _SL, mini)

    h = _SL
    while h > 1:
        h //= 2
        v1, v2 = minv[:h], minv[h:2 * h]
        k1, k2 = mini[:h], mini[h:2 * h]
        take2 = (v2 < v1) | ((v2 == v1) & (k2 < k1))
        minv = jnp.where(take2, v2, v1)
        mini = jnp.where(take2, k2, k1)

    codes_ref[0] = mini

    part = jnp.sum(minv, axis=1, keepdims=True)

    @pl.when(b == 0)
    def _():
        loss_ref[...] = jnp.zeros((1, 1), jnp.float32)

    total = loss_ref[...] + part
    loss_ref[...] = jnp.where(b == B - 1, total * _LOSS_SCALE, total)


def _codes_and_loss(z, embedding):
    loss, codes3 = pl.pallas_call(
        _vq_tc_body,
        grid=(B,),
        in_specs=[
            pl.BlockSpec((1, D, L), lambda b: (b, 0, 0)),
            pl.BlockSpec((N_EMB, D), lambda b: (0, 0)),
        ],
        out_specs=[
            pl.BlockSpec((1, 1), lambda b: (0, 0)),
            pl.BlockSpec((1, 1, L), lambda b: (b, 0, 0)),
        ],
        out_shape=[
            jax.ShapeDtypeStruct((1, 1), jnp.float32),
            jax.ShapeDtypeStruct((B, 1, L), jnp.int32),
        ],
    )(z, embedding)
    return loss[0, 0], codes3.reshape(B, L)


def _sc_gather_body(emb_hbm, codes_hbm, zq_hbm,
                    codes_v, gidx_v, off_v, rows_v, out_v, sem):
    info = plsc.get_sparse_core_info()
    wid = lax.axis_index("s") * info.num_cores + lax.axis_index("c")

    pltpu.sync_copy(codes_hbm.at[wid], codes_v)
    for i in range(L // _NL):
        c16 = codes_v[pl.ds(i * _NL, _NL)]
        gidx_v[pl.ds(i * _NL, _NL)] = c16 >> 1
        off_v[pl.ds(i * _NL, _NL)] = (c16 & 1) << 6

    copies = [
        pltpu.async_copy(
            emb_hbm.at[gidx_v.at[pl.ds(j * _CH, _CH)]],
            rows_v.at[pl.ds(j * _CH, _CH), :], sem.at[j])
        for j in range(L // _CH)
    ]

    lane = lax.iota(jnp.int32, _NL)
    per_chunk = _CH // _NL

    def lc_body(lc, carry):
        row16 = lc * _NL + lane
        cbase = off_v[pl.ds(lc * _NL, _NL)]

        @plsc.parallel_loop(0, D, unroll=16)
        def _(d):
            vals = plsc.load_gather(rows_v, [row16, cbase + d])
            out_v[d, pl.ds(lc * _NL, _NL)] = vals

        return carry

    for j in range(L // _CH):
        copies[j].wait()
        lax.fori_loop(j * per_chunk, (j + 1) * per_chunk, lc_body, 0)

    pltpu.sync_copy(out_v, zq_hbm.at[wid])


def _sc_gather(embedding, codes):
    mesh = plsc.VectorSubcoreMesh(core_axis_name="c", subcore_axis_name="s")
    return pl.kernel(
        _sc_gather_body,
        out_type=jax.ShapeDtypeStruct((B, D, L), jnp.float32),
        mesh=mesh,
        compiler_params=pltpu.CompilerParams(needs_layout_passes=False),
        scratch_types=[
            pltpu.VMEM((L,), jnp.int32),
            pltpu.VMEM((L,), jnp.int32),
            pltpu.VMEM((L,), jnp.int32),
            pltpu.VMEM((L, _CW), jnp.float32),
            pltpu.VMEM((D, L), jnp.float32),
            pltpu.SemaphoreType.DMA((L // _CH,)),
        ],
    )(embedding.reshape(N_EMB // 2, _CW), codes)


@jax.jit
def kernel(z, embedding):
    loss, codes = _codes_and_loss(z, embedding)
    zq = _sc_gather(embedding, codes)
    return zq, loss, codes
